# baseline (speedup 1.0000x reference)
import functools

import jax
import jax.numpy as jnp
from jax import lax
from jax.experimental import pallas as pl
from jax.experimental.pallas import tpu as pltpu
from jax.experimental.pallas import tpu_sc as plsc

_MB = 512
_NCHUNK = 128
_NC = 2
_NS = 16
_LANES = 16


def _dist_argmin_body(kt_ref, pc_ref, idx_ref, nrm_ref):
    b = pl.program_id(0)
    n = pc_ref.shape[2]
    kc = [kt_ref[0, :, c : c + 1] for c in range(3)]
    iota = lax.broadcasted_iota(jnp.int32, (_MB, _NCHUNK), 1)
    run_min = None
    run_chunk = None
    for j in range(n // _NCHUNK):
        sl = pl.ds(j * _NCHUNK, _NCHUNK)
        acc = None
        for c in range(3):
            d = kc[c] - pc_ref[0, c : c + 1, sl]
            acc = d * d if acc is None else acc + d * d
        if run_min is None:
            run_min = acc
            run_chunk = jnp.zeros((_MB, _NCHUNK), jnp.int32)
        else:
            m = acc < run_min
            run_min = jnp.where(m, acc, run_min)
            run_chunk = jnp.where(m, j, run_chunk)
    gmin = jnp.min(run_min, axis=1, keepdims=True)
    cand = jnp.where(run_min == gmin, run_chunk * _NCHUNK + iota, n)
    idx = jnp.min(cand, axis=1)
    idx_ref[0, :] = idx + (3 * n) * b
    nrm_ref[0, :] = jnp.sqrt(gmin[:, 0])


def _dist_argmin(kt, pc):
    B, M, _ = kt.shape
    N = pc.shape[2]
    grid = (B, M // _MB)
    return pl.pallas_call(
        _dist_argmin_body,
        grid=grid,
        in_specs=[
            pl.BlockSpec((1, _MB, 3), lambda b, j: (b, j, 0)),
            pl.BlockSpec((1, 3, N), lambda b, j: (b, 0, 0)),
        ],
        out_specs=[
            pl.BlockSpec((1, _MB), lambda b, j: (0, b * (M // _MB) + j)),
            pl.BlockSpec((1, _MB), lambda b, j: (0, b * (M // _MB) + j)),
        ],
        out_shape=[
            jax.ShapeDtypeStruct((1, B * M), jnp.int32),
            jax.ShapeDtypeStruct((1, B * M), jnp.float32),
        ],
    )(kt, pc)


def _make_sc_gather_loss(total, wpt, M, N):
    mesh = plsc.VectorSubcoreMesh(
        core_axis_name="c", subcore_axis_name="s",
        num_cores=_NC, num_subcores=_NS,
    )

    @functools.partial(
        pl.kernel,
        out_type=jax.ShapeDtypeStruct((total,), jnp.float32),
        mesh=mesh,
        scratch_types=[
            [pltpu.VMEM((wpt,), jnp.int32) for _ in range(3)],
            [pltpu.VMEM((wpt,), jnp.float32) for _ in range(6)],
            [pltpu.VMEM((wpt,), jnp.float32) for _ in range(3)],
            pltpu.VMEM((wpt,), jnp.float32),
            pltpu.VMEM((wpt,), jnp.float32),
            pltpu.SemaphoreType.DMA,
            pltpu.SemaphoreType.DMA,
        ],
    )
    def sc_kernel(pc_hbm, sn_hbm, kp_hbm, idx_hbm, nrm_hbm, out_hbm,
                  idx_v, gat_v, k_v, nrm_v, loss_v, sem, sem_idx):
        wid = lax.axis_index("s") * _NC + lax.axis_index("c")
        base = wid * wpt
        cp_idx = pltpu.async_copy(idx_hbm.at[pl.ds(base, wpt)], idx_v[0],
                                  sem_idx)
        copies = [pltpu.async_copy(nrm_hbm.at[pl.ds(base, wpt)], nrm_v, sem)]
        b = base // M
        m0 = base - b * M
        for c in range(3):
            copies.append(pltpu.async_copy(
                kp_hbm.at[pl.ds(b * 3 * M + c * M + m0, wpt)], k_v[c], sem))
        cp_idx.wait()
        for g in range(wpt // _LANES):
            sl = pl.ds(g * _LANES, _LANES)
            v = idx_v[0][sl]
            idx_v[1][sl] = v + N
            idx_v[2][sl] = v + 2 * N
        copies += [
            pltpu.async_copy(src.at[idx_v[c]], gat_v[3 * s + c], sem)
            for s, src in enumerate((pc_hbm, sn_hbm))
            for c in range(3)
        ]
        for cp in copies:
            cp.wait()
        for g in range(wpt // _LANES):
            sl = pl.ds(g * _LANES, _LANES)
            p0, p1, p2 = gat_v[0][sl], gat_v[1][sl], gat_v[2][sl]
            s0, s1, s2 = gat_v[3][sl], gat_v[4][sl], gat_v[5][sl]
            inv = 1.0 / (nrm_v[sl] + 1e-7)
            t0 = (k_v[0][sl] - p0) * inv
            t1 = (k_v[1][sl] - p1) * inv
            t2 = (k_v[2][sl] - p2) * inv
            dot = s0 * t0 + s1 * t1 + s2 * t2
            loss_v[sl] = dot * dot
        pltpu.sync_copy(loss_v, out_hbm.at[pl.ds(base, wpt)])

    return sc_kernel


def kernel(keypoint, pc, sn):
    B, _, M = keypoint.shape
    N = pc.shape[2]
    kt = jnp.transpose(keypoint, (0, 2, 1))

    BH = B // 2
    total = BH * M
    wpt = total // (_NC * _NS)
    sck = _make_sc_gather_loss(total, wpt, M, N)
    losses = []
    for h in range(2):
        bs = slice(h * BH, (h + 1) * BH)
        idxf, nrmf = _dist_argmin(kt[bs], pc[bs])
        losses.append(sck(pc[bs].reshape(-1), sn[bs].reshape(-1),
                          keypoint[bs].reshape(-1),
                          idxf.reshape(-1), nrmf.reshape(-1)))
    return jnp.concatenate(losses).reshape(B, M, 1, 1)


# speedup vs baseline: 1.1338x; 1.1338x over previous
import functools

import jax
import jax.numpy as jnp
from jax import lax
from jax.experimental import pallas as pl
from jax.experimental.pallas import tpu as pltpu
from jax.experimental.pallas import tpu_sc as plsc

_NCHUNK = 128
_NC = 2
_NS = 16
_LANES = 16


def _dist_argmin_body(kp_ref, pc_ref, sn_ref, idx_ref, nrm_ref,
                      k0_ref, k1_ref, k2_ref,
                      p0_ref, p1_ref, p2_ref,
                      s0_ref, s1_ref, s2_ref):
    b = pl.program_id(0)
    n = pc_ref.shape[2]
    m = kp_ref.shape[2]
    kt = jnp.transpose(kp_ref[0], (1, 0))
    kc = [kt[:, c : c + 1] for c in range(3)]
    iota = lax.broadcasted_iota(jnp.int32, (m, _NCHUNK), 1)
    run_min = None
    run_chunk = None
    for j in range(n // _NCHUNK):
        sl = pl.ds(j * _NCHUNK, _NCHUNK)
        acc = None
        for c in range(3):
            d = kc[c] - pc_ref[0, c : c + 1, sl]
            acc = d * d if acc is None else acc + d * d
        if run_min is None:
            run_min = acc
            run_chunk = jnp.zeros((m, _NCHUNK), jnp.int32)
        else:
            mask = acc < run_min
            run_min = jnp.where(mask, acc, run_min)
            run_chunk = jnp.where(mask, j, run_chunk)
    gmin = jnp.min(run_min, axis=1, keepdims=True)
    cand = jnp.where(run_min == gmin, run_chunk * _NCHUNK + iota, n)
    idx = jnp.min(cand, axis=1)
    idx_ref[0, :] = idx + n * b
    nrm_ref[0, :] = jnp.sqrt(gmin[:, 0])
    for c, r in enumerate((k0_ref, k1_ref, k2_ref)):
        r[0, :] = kp_ref[0, c, :]
    for c, r in enumerate((p0_ref, p1_ref, p2_ref)):
        r[0, :] = pc_ref[0, c, :]
    for c, r in enumerate((s0_ref, s1_ref, s2_ref)):
        r[0, :] = sn_ref[0, c, :]


def _dist_argmin(kp, pc, sn):
    B, _, M = kp.shape
    N = pc.shape[2]
    m_spec = pl.BlockSpec((1, M), lambda b: (0, b))
    n_spec = pl.BlockSpec((1, N), lambda b: (0, b))
    return pl.pallas_call(
        _dist_argmin_body,
        grid=(B,),
        in_specs=[
            pl.BlockSpec((1, 3, M), lambda b: (b, 0, 0)),
            pl.BlockSpec((1, 3, N), lambda b: (b, 0, 0)),
            pl.BlockSpec((1, 3, N), lambda b: (b, 0, 0)),
        ],
        out_specs=[m_spec, m_spec, m_spec, m_spec, m_spec,
                   n_spec, n_spec, n_spec, n_spec, n_spec, n_spec],
        out_shape=[
            jax.ShapeDtypeStruct((1, B * M), jnp.int32),
            jax.ShapeDtypeStruct((1, B * M), jnp.float32),
            jax.ShapeDtypeStruct((1, B * M), jnp.float32),
            jax.ShapeDtypeStruct((1, B * M), jnp.float32),
            jax.ShapeDtypeStruct((1, B * M), jnp.float32),
            jax.ShapeDtypeStruct((1, B * N), jnp.float32),
            jax.ShapeDtypeStruct((1, B * N), jnp.float32),
            jax.ShapeDtypeStruct((1, B * N), jnp.float32),
            jax.ShapeDtypeStruct((1, B * N), jnp.float32),
            jax.ShapeDtypeStruct((1, B * N), jnp.float32),
            jax.ShapeDtypeStruct((1, B * N), jnp.float32),
        ],
    )(kp, pc, sn)


def _make_sc_gather_loss(total, wpt):
    mesh = plsc.VectorSubcoreMesh(
        core_axis_name="c", subcore_axis_name="s",
        num_cores=_NC, num_subcores=_NS,
    )

    @functools.partial(
        pl.kernel,
        out_type=jax.ShapeDtypeStruct((total,), jnp.float32),
        mesh=mesh,
        scratch_types=[
            pltpu.VMEM((wpt,), jnp.int32),
            [pltpu.VMEM((wpt,), jnp.float32) for _ in range(6)],
            [pltpu.VMEM((wpt,), jnp.float32) for _ in range(3)],
            pltpu.VMEM((wpt,), jnp.float32),
            pltpu.VMEM((wpt,), jnp.float32),
            pltpu.SemaphoreType.DMA,
            pltpu.SemaphoreType.DMA,
        ],
    )
    def sc_kernel(p0_hbm, p1_hbm, p2_hbm, s0_hbm, s1_hbm, s2_hbm,
                  k0_hbm, k1_hbm, k2_hbm, idx_hbm, nrm_hbm, out_hbm,
                  idx_v, gat_v, k_v, nrm_v, loss_v, sem, sem_idx):
        wid = lax.axis_index("s") * _NC + lax.axis_index("c")
        base = wid * wpt
        cp_idx = pltpu.async_copy(idx_hbm.at[pl.ds(base, wpt)], idx_v,
                                  sem_idx)
        copies = [pltpu.async_copy(nrm_hbm.at[pl.ds(base, wpt)], nrm_v, sem)]
        for c, src in enumerate((k0_hbm, k1_hbm, k2_hbm)):
            copies.append(pltpu.async_copy(
                src.at[pl.ds(base, wpt)], k_v[c], sem))
        cp_idx.wait()
        copies += [
            pltpu.async_copy(src.at[idx_v], gat_v[i], sem)
            for i, src in enumerate(
                (p0_hbm, p1_hbm, p2_hbm, s0_hbm, s1_hbm, s2_hbm))
        ]
        for cp in copies:
            cp.wait()
        for g in range(wpt // _LANES):
            sl = pl.ds(g * _LANES, _LANES)
            p0, p1, p2 = gat_v[0][sl], gat_v[1][sl], gat_v[2][sl]
            s0, s1, s2 = gat_v[3][sl], gat_v[4][sl], gat_v[5][sl]
            inv = 1.0 / (nrm_v[sl] + 1e-7)
            t0 = (k_v[0][sl] - p0) * inv
            t1 = (k_v[1][sl] - p1) * inv
            t2 = (k_v[2][sl] - p2) * inv
            dot = s0 * t0 + s1 * t1 + s2 * t2
            loss_v[sl] = dot * dot
        pltpu.sync_copy(loss_v, out_hbm.at[pl.ds(base, wpt)])

    return sc_kernel


def kernel(keypoint, pc, sn):
    B, _, M = keypoint.shape
    N = pc.shape[2]
    outs = _dist_argmin(keypoint, pc, sn)
    idxf, nrmf, k0, k1, k2, p0, p1, p2, s0, s1, s2 = [
        o.reshape(-1) for o in outs]
    total = B * M
    wpt = total // (_NC * _NS)
    sck = _make_sc_gather_loss(total, wpt)
    loss = sck(p0, p1, p2, s0, s1, s2, k0, k1, k2, idxf, nrmf)
    return loss.reshape(B, M, 1, 1)


# speedup vs baseline: 1.2757x; 1.1252x over previous
import functools

import jax
import jax.numpy as jnp
from jax import lax
from jax.experimental import pallas as pl
from jax.experimental.pallas import tpu as pltpu
from jax.experimental.pallas import tpu_sc as plsc

_NCHUNK = 128
_NC = 2
_NS = 16
_LANES = 16


def _dist_argmin_body(kp_ref, pc_ref, sn_ref, idx_ref, nrm_ref,
                      k0_ref, k1_ref, k2_ref,
                      p0_ref, p1_ref, p2_ref,
                      s0_ref, s1_ref, s2_ref):
    n = pc_ref.shape[2]
    m = kp_ref.shape[2]
    B = kp_ref.shape[1]
    iota = lax.broadcasted_iota(jnp.int32, (m, _NCHUNK), 1)
    for b in range(B):
        kt = jnp.transpose(kp_ref[:, b, :], (1, 0))
        kc = [kt[:, c : c + 1] for c in range(3)]
        run_min = None
        run_chunk = None
        for j in range(n // _NCHUNK):
            sl = pl.ds(j * _NCHUNK, _NCHUNK)
            acc = None
            for c in range(3):
                d = kc[c] - pc_ref[c, b : b + 1, sl]
                acc = d * d if acc is None else acc + d * d
            if run_min is None:
                run_min = acc
                run_chunk = jnp.zeros((m, _NCHUNK), jnp.int32)
            else:
                mask = acc < run_min
                run_min = jnp.where(mask, acc, run_min)
                run_chunk = jnp.where(mask, j, run_chunk)
        gmin = jnp.min(run_min, axis=1, keepdims=True)
        cand = jnp.where(run_min == gmin, run_chunk * _NCHUNK + iota, n)
        idx = jnp.min(cand, axis=1)
        msl = pl.ds(b * m, m)
        nsl = pl.ds(b * n, n)
        idx_ref[0, msl] = idx + n * b
        nrm_ref[0, msl] = jnp.sqrt(gmin[:, 0])
        for c, r in enumerate((k0_ref, k1_ref, k2_ref)):
            r[0, msl] = kp_ref[c, b, :]
        for c, r in enumerate((p0_ref, p1_ref, p2_ref)):
            r[0, nsl] = pc_ref[c, b, :]
        for c, r in enumerate((s0_ref, s1_ref, s2_ref)):
            r[0, nsl] = sn_ref[c, b, :]


def _dist_argmin(kp, pc, sn):
    _, B, M = kp.shape
    N = pc.shape[2]
    m_spec = pl.BlockSpec((1, B * M), lambda: (0, 0))
    n_spec = pl.BlockSpec((1, B * N), lambda: (0, 0))
    return pl.pallas_call(
        _dist_argmin_body,
        grid=(),
        in_specs=[
            pl.BlockSpec((3, B, M), lambda: (0, 0, 0)),
            pl.BlockSpec((3, B, N), lambda: (0, 0, 0)),
            pl.BlockSpec((3, B, N), lambda: (0, 0, 0)),
        ],
        out_specs=[m_spec, m_spec, m_spec, m_spec, m_spec,
                   n_spec, n_spec, n_spec, n_spec, n_spec, n_spec],
        out_shape=[
            jax.ShapeDtypeStruct((1, B * M), jnp.int32),
            jax.ShapeDtypeStruct((1, B * M), jnp.float32),
            jax.ShapeDtypeStruct((1, B * M), jnp.float32),
            jax.ShapeDtypeStruct((1, B * M), jnp.float32),
            jax.ShapeDtypeStruct((1, B * M), jnp.float32),
            jax.ShapeDtypeStruct((1, B * N), jnp.float32),
            jax.ShapeDtypeStruct((1, B * N), jnp.float32),
            jax.ShapeDtypeStruct((1, B * N), jnp.float32),
            jax.ShapeDtypeStruct((1, B * N), jnp.float32),
            jax.ShapeDtypeStruct((1, B * N), jnp.float32),
            jax.ShapeDtypeStruct((1, B * N), jnp.float32),
        ],
    )(kp, pc, sn)


def _make_sc_gather_loss(total, wpt):
    mesh = plsc.VectorSubcoreMesh(
        core_axis_name="c", subcore_axis_name="s",
        num_cores=_NC, num_subcores=_NS,
    )

    @functools.partial(
        pl.kernel,
        out_type=jax.ShapeDtypeStruct((total,), jnp.float32),
        mesh=mesh,
        scratch_types=[
            pltpu.VMEM((wpt,), jnp.int32),
            [pltpu.VMEM((wpt,), jnp.float32) for _ in range(6)],
            [pltpu.VMEM((wpt,), jnp.float32) for _ in range(3)],
            pltpu.VMEM((wpt,), jnp.float32),
            pltpu.VMEM((wpt,), jnp.float32),
            pltpu.SemaphoreType.DMA,
            pltpu.SemaphoreType.DMA,
        ],
    )
    def sc_kernel(p0_hbm, p1_hbm, p2_hbm, s0_hbm, s1_hbm, s2_hbm,
                  k0_hbm, k1_hbm, k2_hbm, idx_hbm, nrm_hbm, out_hbm,
                  idx_v, gat_v, k_v, nrm_v, loss_v, sem, sem_idx):
        wid = lax.axis_index("s") * _NC + lax.axis_index("c")
        base = wid * wpt
        cp_idx = pltpu.async_copy(idx_hbm.at[pl.ds(base, wpt)], idx_v,
                                  sem_idx)
        copies = [pltpu.async_copy(nrm_hbm.at[pl.ds(base, wpt)], nrm_v, sem)]
        for c, src in enumerate((k0_hbm, k1_hbm, k2_hbm)):
            copies.append(pltpu.async_copy(
                src.at[pl.ds(base, wpt)], k_v[c], sem))
        cp_idx.wait()
        copies += [
            pltpu.async_copy(src.at[idx_v], gat_v[i], sem)
            for i, src in enumerate(
                (p0_hbm, p1_hbm, p2_hbm, s0_hbm, s1_hbm, s2_hbm))
        ]
        for cp in copies:
            cp.wait()
        for g in range(wpt // _LANES):
            sl = pl.ds(g * _LANES, _LANES)
            p0, p1, p2 = gat_v[0][sl], gat_v[1][sl], gat_v[2][sl]
            s0, s1, s2 = gat_v[3][sl], gat_v[4][sl], gat_v[5][sl]
            inv = 1.0 / (nrm_v[sl] + 1e-7)
            t0 = (k_v[0][sl] - p0) * inv
            t1 = (k_v[1][sl] - p1) * inv
            t2 = (k_v[2][sl] - p2) * inv
            dot = s0 * t0 + s1 * t1 + s2 * t2
            loss_v[sl] = dot * dot
        pltpu.sync_copy(loss_v, out_hbm.at[pl.ds(base, wpt)])

    return sc_kernel


def kernel(keypoint, pc, sn):
    B, _, M = keypoint.shape
    N = pc.shape[2]
    outs = _dist_argmin(jnp.transpose(keypoint, (1, 0, 2)),
                        jnp.transpose(pc, (1, 0, 2)),
                        jnp.transpose(sn, (1, 0, 2)))
    idxf, nrmf, k0, k1, k2, p0, p1, p2, s0, s1, s2 = [
        o.reshape(-1) for o in outs]
    total = B * M
    wpt = total // (_NC * _NS)
    sck = _make_sc_gather_loss(total, wpt)
    loss = sck(p0, p1, p2, s0, s1, s2, k0, k1, k2, idxf, nrmf)
    return loss.reshape(B, M, 1, 1)
